# baseline (speedup 1.0000x reference)
import functools

import jax
import jax.numpy as jnp
from jax import lax
from jax.experimental import pallas as pl
from jax.experimental.pallas import tpu as pltpu
from jax.experimental.pallas import tpu_sc as plsc

L = 16
VB = 256
DB = 400
FB = 2096
NC = 2
NS = 16
NW = NC * NS


def _sc_interp(v_t, f_t, face_ids, bary_t, nbatch):
    nplanes, n_src = v_t.shape
    ncomp = nplanes // nbatch
    n_dst = face_ids.shape[0]
    nvb = (n_src + VB - 1) // VB
    nfb = (n_dst + DB - 1) // DB
    n_faces = f_t.shape[1]
    nsb = (n_faces + FB - 1) // FB
    mesh = plsc.VectorSubcoreMesh(core_axis_name="c", subcore_axis_name="s")

    @functools.partial(
        pl.kernel,
        mesh=mesh,
        compiler_params=pltpu.CompilerParams(
            needs_layout_passes=False, use_tc_tiling_on_sc=False),
        out_type=jax.ShapeDtypeStruct((nplanes, n_dst), jnp.float32),
        scratch_types=[
            pltpu.VMEM_SHARED((n_src, 16), jnp.float32),
            pltpu.VMEM_SHARED((n_faces,), jnp.int32),
            pltpu.VMEM_SHARED((n_faces,), jnp.int32),
            pltpu.VMEM_SHARED((n_faces,), jnp.int32),
            pltpu.VMEM((12, VB), jnp.float32),
            pltpu.VMEM((VB, 16), jnp.float32),
            pltpu.VMEM((DB,), jnp.int32),
            pltpu.VMEM((DB,), jnp.int32),
            pltpu.VMEM((DB,), jnp.int32),
            pltpu.VMEM((DB,), jnp.int32),
            pltpu.VMEM((DB, 16), jnp.float32),
            pltpu.VMEM((DB, 16), jnp.float32),
            pltpu.VMEM((DB, 16), jnp.float32),
            pltpu.VMEM((4, DB), jnp.float32),
            pltpu.VMEM((12, DB), jnp.float32),
            pltpu.VMEM((DB,), jnp.int32),
            pltpu.VMEM((DB,), jnp.int32),
            pltpu.VMEM((DB,), jnp.int32),
            pltpu.VMEM((DB,), jnp.int32),
            pltpu.VMEM((DB, 16), jnp.float32),
            pltpu.VMEM((DB, 16), jnp.float32),
            pltpu.VMEM((DB, 16), jnp.float32),
            pltpu.VMEM((4, DB), jnp.float32),
            pltpu.VMEM((12, DB), jnp.float32),
            pltpu.SemaphoreType.DMA,
            pltpu.SemaphoreType.DMA,
        ],
    )
    def k(v_hbm, f_hbm, fid_hbm, bary_hbm, out_hbm,
          vt_sp, f0_sp, f1_sp, f2_sp,
          vin_v, pack_v, fid_v, i0_v, i1_v, i2_v,
          v0_v, v1_v, v2_v, wb_v, out_v,
          fid_b, i0_b, i1_b, i2_b, v0_b, v1_b, v2_b, wb_b, out_b,
          sem, sem_b):
        cid = lax.axis_index("c")
        sid = lax.axis_index("s")
        wid = sid * NC + cid
        iota = lax.iota(jnp.int32, L)

        def vbuild(i, carry):
            blkid = sid + i * NS
            base = jnp.minimum(blkid * VB, n_src - VB)
            cps = [pltpu.async_copy(v_hbm.at[j, pl.ds(base, VB)],
                                    vin_v.at[j], sem)
                   for j in range(12)]
            for cp in cps:
                cp.wait()

            def g_body(g, c2):
                row = g * L + iota
                for c in range(ncomp):
                    for r in range(nbatch):
                        x = vin_v[c * nbatch + r, pl.ds(g * L, L)]
                        plsc.store_scatter(
                            pack_v,
                            [row, jnp.full((L,), r * 3 + c, jnp.int32)], x)
                return c2
            lax.fori_loop(0, VB // L, g_body, 0)
            pltpu.sync_copy(pack_v, vt_sp.at[pl.ds(base, VB)])
            return carry
        lax.fori_loop(0, (nvb - 1 - sid) // NS + 1, vbuild, 0)

        def fbuild(i, carry):
            blkid = sid + i * NS
            base = jnp.minimum(blkid * FB, n_faces - FB)
            cps = [pltpu.async_copy(f_hbm.at[c, pl.ds(base, FB)],
                                    dst.at[pl.ds(base, FB)], sem)
                   for c, dst in ((0, f0_sp), (1, f1_sp), (2, f2_sp))]
            for cp in cps:
                cp.wait()
            return carry
        lax.fori_loop(0, (nsb - 1 - sid) // NS + 1, fbuild, 0)

        plsc.subcore_barrier()

        chainA = (fid_v, i0_v, i1_v, i2_v, v0_v, v1_v, v2_v, wb_v, out_v, sem)
        chainB = (fid_b, i0_b, i1_b, i2_b, v0_b, v1_b, v2_b, wb_b, out_b,
                  sem_b)

        def fetch_in(base, ch):
            fid, _, _, _, _, _, _, wb, _, sm = ch
            cps = [pltpu.async_copy(fid_hbm.at[pl.ds(base, DB)], fid, sm)]
            cps += [pltpu.async_copy(bary_hbm.at[kk, pl.ds(base, DB)],
                                     wb.at[kk], sm) for kk in range(4)]
            return cps

        def fetch_idx(ch):
            fid, i0, i1, i2, _, _, _, _, _, sm = ch
            return [pltpu.async_copy(f0_sp.at[fid], i0, sm),
                    pltpu.async_copy(f1_sp.at[fid], i1, sm),
                    pltpu.async_copy(f2_sp.at[fid], i2, sm)]

        def fetch_vert(ch):
            _, i0, i1, i2, v0, v1, v2, _, _, sm = ch
            return [pltpu.async_copy(vt_sp.at[i0], v0, sm),
                    pltpu.async_copy(vt_sp.at[i1], v1, sm),
                    pltpu.async_copy(vt_sp.at[i2], v2, sm)]

        def compute(ch):
            _, _, _, _, v0_r, v1_r, v2_r, wb_r, out_r, _ = ch

            def compute_body(g, c2):
                row = g * L + iota

                def ld(ref, c):
                    return plsc.load_gather(
                        ref, [row, jnp.full((L,), c, jnp.int32)])

                sl = pl.ds(g * L, L)
                w0, w1, w2, w3 = (wb_r[kk, sl] for kk in range(4))
                w03 = w0 + w3
                for r in range(nbatch):
                    c0 = r * 3
                    v0x, v0y, v0z = ld(v0_r, c0), ld(v0_r, c0 + 1), ld(v0_r, c0 + 2)
                    v1x, v1y, v1z = ld(v1_r, c0), ld(v1_r, c0 + 1), ld(v1_r, c0 + 2)
                    v2x, v2y, v2z = ld(v2_r, c0), ld(v2_r, c0 + 1), ld(v2_r, c0 + 2)
                    e1x, e1y, e1z = v1x - v0x, v1y - v0y, v1z - v0z
                    e2x, e2y, e2z = v2x - v0x, v2y - v0y, v2z - v0z
                    cx = e1y * e2z - e1z * e2y
                    cy = e1z * e2x - e1x * e2z
                    cz = e1x * e2y - e1y * e2x
                    rx = w03 * v0x + w1 * v1x + w2 * v2x + w3 * cx
                    ry = w03 * v0y + w1 * v1y + w2 * v2y + w3 * cy
                    rz = w03 * v0z + w1 * v1z + w2 * v2z + w3 * cz
                    for c, rr in ((0, rx), (1, ry), (2, rz)):
                        plsc.store_scatter(
                            out_r,
                            [jnp.full((L,), c * nbatch + r, jnp.int32), row],
                            rr)
                return c2
            lax.fori_loop(0, DB // L, compute_body, 0)

        def store_out(base, ch):
            out_r, sm = ch[8], ch[9]
            return [pltpu.async_copy(out_r.at[j],
                                     out_hbm.at[j, pl.ds(base, DB)], sm)
                    for j in range(12)]

        def wait_all(cps):
            for cp in cps:
                cp.wait()

        def block_pair(t, carry):
            base_a = jnp.minimum((wid + (2 * t) * NW) * DB, n_dst - DB)
            base_b = jnp.minimum((wid + (2 * t + 1) * NW) * DB, n_dst - DB)
            in_a = fetch_in(base_a, chainA)
            in_b = fetch_in(base_b, chainB)
            wait_all(in_a)
            idx_a = fetch_idx(chainA)
            wait_all(in_b)
            idx_b = fetch_idx(chainB)
            wait_all(idx_a)
            vert_a = fetch_vert(chainA)
            wait_all(idx_b)
            vert_b = fetch_vert(chainB)
            wait_all(vert_a)
            compute(chainA)
            out_a = store_out(base_a, chainA)
            wait_all(vert_b)
            compute(chainB)
            out_b = store_out(base_b, chainB)
            wait_all(out_a)
            wait_all(out_b)
            return carry

        lax.fori_loop(0, (nfb + 2 * NW - 1) // (2 * NW), block_pair, 0)

    return k(v_t, f_t, face_ids, bary_t)


def kernel(V_src_deformed, F_src, face_ids, bary_coords):
    nbatch, n_src, ncomp = V_src_deformed.shape
    n_dst = face_ids.shape[0]
    v_t = jnp.transpose(V_src_deformed, (2, 0, 1)).reshape(
        ncomp * nbatch, n_src)
    f_t = jnp.transpose(F_src, (1, 0))
    bary_t = jnp.transpose(bary_coords, (1, 0))
    out = _sc_interp(v_t, f_t, face_ids, bary_t, nbatch)
    out = out.reshape(ncomp, nbatch, n_dst)
    return jnp.transpose(out, (1, 2, 0))


# speedup vs baseline: 1.0302x; 1.0302x over previous
import functools

import jax
import jax.numpy as jnp
from jax import lax
from jax.experimental import pallas as pl
from jax.experimental.pallas import tpu as pltpu
from jax.experimental.pallas import tpu_sc as plsc

L = 16
VB = 528
DB = 640
FB = 2096
NC = 2
NS = 16
NW = NC * NS


def _sc_interp(v_t, f_t, face_ids, bary_t, nbatch):
    nplanes, n_src = v_t.shape
    ncomp = nplanes // nbatch
    n_dst = face_ids.shape[0]
    nvb = (n_src + VB - 1) // VB
    nfb = (n_dst + DB - 1) // DB
    n_faces = f_t.shape[1]
    nsb = (n_faces + FB - 1) // FB
    mesh = plsc.VectorSubcoreMesh(core_axis_name="c", subcore_axis_name="s")

    @functools.partial(
        pl.kernel,
        mesh=mesh,
        compiler_params=pltpu.CompilerParams(
            needs_layout_passes=False, use_tc_tiling_on_sc=False),
        out_type=jax.ShapeDtypeStruct((nplanes, n_dst), jnp.float32),
        scratch_types=[
            pltpu.VMEM_SHARED((n_src, 16), jnp.float32),
            pltpu.VMEM_SHARED((n_faces,), jnp.int32),
            pltpu.VMEM_SHARED((n_faces,), jnp.int32),
            pltpu.VMEM_SHARED((n_faces,), jnp.int32),
            pltpu.VMEM((12, VB), jnp.float32),
            pltpu.VMEM((VB, 16), jnp.float32),
            pltpu.VMEM((DB,), jnp.int32),
            pltpu.VMEM((DB,), jnp.int32),
            pltpu.VMEM((DB,), jnp.int32),
            pltpu.VMEM((DB,), jnp.int32),
            pltpu.VMEM((DB, 16), jnp.float32),
            pltpu.VMEM((DB, 16), jnp.float32),
            pltpu.VMEM((DB, 16), jnp.float32),
            pltpu.VMEM((4, DB), jnp.float32),
            pltpu.VMEM((12, DB), jnp.float32),
            pltpu.SemaphoreType.DMA,
        ],
    )
    def k(v_hbm, f_hbm, fid_hbm, bary_hbm, out_hbm,
          vt_sp, f0_sp, f1_sp, f2_sp,
          vin_v, pack_v, fid_v, i0_v, i1_v, i2_v,
          v0_v, v1_v, v2_v, wb_v, out_v, sem):
        cid = lax.axis_index("c")
        sid = lax.axis_index("s")
        wid = sid * NC + cid
        iota = lax.iota(jnp.int32, L)

        def vbuild(i, carry):
            blkid = sid + i * NS
            base = jnp.minimum(blkid * VB, n_src - VB)
            cps = [pltpu.async_copy(v_hbm.at[j, pl.ds(base, VB)],
                                    vin_v.at[j], sem)
                   for j in range(12)]
            for cp in cps:
                cp.wait()

            def g_body(g, c2):
                row = g * L + iota
                for c in range(ncomp):
                    for r in range(nbatch):
                        x = vin_v[c * nbatch + r, pl.ds(g * L, L)]
                        plsc.store_scatter(
                            pack_v,
                            [row, jnp.full((L,), r * 3 + c, jnp.int32)], x)
                return c2
            lax.fori_loop(0, VB // L, g_body, 0)
            pltpu.sync_copy(pack_v, vt_sp.at[pl.ds(base, VB)])
            return carry
        lax.fori_loop(0, (nvb - 1 - sid) // NS + 1, vbuild, 0)

        def fbuild(i, carry):
            blkid = sid + i * NS
            base = jnp.minimum(blkid * FB, n_faces - FB)
            cps = [pltpu.async_copy(f_hbm.at[c, pl.ds(base, FB)],
                                    dst.at[pl.ds(base, FB)], sem)
                   for c, dst in ((0, f0_sp), (1, f1_sp), (2, f2_sp))]
            for cp in cps:
                cp.wait()
            return carry
        lax.fori_loop(0, (nsb - 1 - sid) // NS + 1, fbuild, 0)

        plsc.subcore_barrier()

        def block_body(i, carry):
            blkid = wid + i * NW
            base = jnp.minimum(blkid * DB, n_dst - DB)
            cps = [pltpu.async_copy(fid_hbm.at[pl.ds(base, DB)], fid_v, sem)]
            cps += [pltpu.async_copy(bary_hbm.at[kk, pl.ds(base, DB)],
                                     wb_v.at[kk], sem)
                    for kk in range(4)]
            for cp in cps:
                cp.wait()
            cps = [pltpu.async_copy(f0_sp.at[fid_v], i0_v, sem),
                   pltpu.async_copy(f1_sp.at[fid_v], i1_v, sem),
                   pltpu.async_copy(f2_sp.at[fid_v], i2_v, sem)]
            for cp in cps:
                cp.wait()
            cps = [pltpu.async_copy(vt_sp.at[i0_v], v0_v, sem),
                   pltpu.async_copy(vt_sp.at[i1_v], v1_v, sem),
                   pltpu.async_copy(vt_sp.at[i2_v], v2_v, sem)]
            for cp in cps:
                cp.wait()

            def compute_body(g, c2):
                row = g * L + iota

                def ld(ref, c):
                    return plsc.load_gather(
                        ref, [row, jnp.full((L,), c, jnp.int32)])

                sl = pl.ds(g * L, L)
                w0, w1, w2, w3 = (wb_v[kk, sl] for kk in range(4))
                w03 = w0 + w3
                for r in range(nbatch):
                    c0 = r * 3
                    v0x, v0y, v0z = ld(v0_v, c0), ld(v0_v, c0 + 1), ld(v0_v, c0 + 2)
                    v1x, v1y, v1z = ld(v1_v, c0), ld(v1_v, c0 + 1), ld(v1_v, c0 + 2)
                    v2x, v2y, v2z = ld(v2_v, c0), ld(v2_v, c0 + 1), ld(v2_v, c0 + 2)
                    e1x, e1y, e1z = v1x - v0x, v1y - v0y, v1z - v0z
                    e2x, e2y, e2z = v2x - v0x, v2y - v0y, v2z - v0z
                    cx = e1y * e2z - e1z * e2y
                    cy = e1z * e2x - e1x * e2z
                    cz = e1x * e2y - e1y * e2x
                    rx = w03 * v0x + w1 * v1x + w2 * v2x + w3 * cx
                    ry = w03 * v0y + w1 * v1y + w2 * v2y + w3 * cy
                    rz = w03 * v0z + w1 * v1z + w2 * v2z + w3 * cz
                    for c, rr in ((0, rx), (1, ry), (2, rz)):
                        plsc.store_scatter(
                            out_v,
                            [jnp.full((L,), c * nbatch + r, jnp.int32), row],
                            rr)
                return c2
            lax.fori_loop(0, DB // L, compute_body, 0)

            cps = [pltpu.async_copy(out_v.at[j],
                                    out_hbm.at[j, pl.ds(base, DB)], sem)
                   for j in range(12)]
            for cp in cps:
                cp.wait()
            return carry

        lax.fori_loop(0, (nfb - 1 - wid) // NW + 1, block_body, 0)

    return k(v_t, f_t, face_ids, bary_t)


def kernel(V_src_deformed, F_src, face_ids, bary_coords):
    nbatch, n_src, ncomp = V_src_deformed.shape
    n_dst = face_ids.shape[0]
    v_t = jnp.transpose(V_src_deformed, (2, 0, 1)).reshape(
        ncomp * nbatch, n_src)
    f_t = jnp.transpose(F_src, (1, 0))
    bary_t = jnp.transpose(bary_coords, (1, 0))
    out = _sc_interp(v_t, f_t, face_ids, bary_t, nbatch)
    out = out.reshape(ncomp, nbatch, n_dst)
    return jnp.transpose(out, (1, 2, 0))
